# baseline (speedup 1.0000x reference)
import functools

import jax
import jax.numpy as jnp
from jax import lax
from jax.experimental import pallas as pl
from jax.experimental.pallas import tpu as pltpu
from jax.experimental.pallas import tpu_sc as plsc

_MUL = 16
_NIR = 3
_DIM = 9
_KIDX = (0, 1, 1, 1, 2, 2, 2, 2, 2)
_OUTW = _MUL * _DIM
_WW = _MUL * _NIR
_LANES = 16
_NC = 2
_NS = 16
_NW = _NC * _NS
_CHUNK = 80
_GRP = _CHUNK // _LANES
_AROW = _LANES * _DIM
_NBUF = 4


def _sc_body(n_chunks, a_hbm, w_hbm, o_hbm,
             a_v0, a_v1, a_v2, a_v3,
             w_v0, w_v1, w_v2, w_v3,
             o_v0, o_v1, o_v2, o_v3,
             si0, si1, si2, si3,
             so0, so1, so2, so3):
  wid = lax.axis_index("s") * _NC + lax.axis_index("c")
  cbase = wid * n_chunks
  A = (a_v0, a_v1, a_v2, a_v3)
  W = (w_v0, w_v1, w_v2, w_v3)
  O = (o_v0, o_v1, o_v2, o_v3)
  SI = (si0, si1, si2, si3)
  SO = (so0, so1, so2, so3)

  lane = lax.iota(jnp.int32, _LANES)
  l3 = lane * _NIR
  l9 = lane * _DIM
  l3k = [l3 + k for k in range(_NIR)]
  l9d = [l9 + dd for dd in range(_DIM)]

  half = _CHUNK // 2

  def in_copies(b, t):
    g0 = (cbase + t) * _GRP
    row0 = (cbase + t) * _CHUNK
    return (
        pltpu.make_async_copy(a_hbm.at[pl.ds(g0, _GRP)], A[b], SI[b]),
        pltpu.make_async_copy(w_hbm.at[pl.ds(row0, half)],
                              W[b].at[pl.ds(0, half)], SI[b]),
        pltpu.make_async_copy(w_hbm.at[pl.ds(row0 + half, half)],
                              W[b].at[pl.ds(half, half)], SI[b]),
    )

  def out_copies(b, t):
    row0 = (cbase + t) * _CHUNK
    return [
        pltpu.make_async_copy(
            O[b].at[pl.ds(j * _LANES, _LANES)],
            o_hbm.at[pl.ds(row0 + j * _LANES, _LANES)], SO[b])
        for j in range(_GRP)
    ]

  def start_in(b, t):
    for c in in_copies(b, t):
      c.start()

  def wait_in(b, t):
    for c in in_copies(b, t):
      c.wait()

  def start_out(b, t):
    for c in out_copies(b, t):
      c.start()

  def wait_out(b, t):
    for c in out_copies(b, t):
      c.wait()

  def compute(b):
    a_v, w_v, o_v = A[b], W[b], O[b]

    @plsc.parallel_loop(0, _GRP)
    def group(g):
      for r0 in range(_LANES):
        row = g * _LANES + r0
        rowb = jnp.full((_LANES,), row, jnp.int32)
        if r0 < _LANES - 1:
          av16 = a_v[g, pl.ds(r0 * _DIM, _LANES)]
          sh = 0
        else:
          av16 = a_v[g, pl.ds(_AROW - _LANES, _LANES)]
          sh = r0 * _DIM - (_AROW - _LANES)
        wvs = [plsc.load_gather(w_v, [rowb, l3k[_KIDX[dd]]])
               for dd in range(_DIM)]
        prods = [wv * av16[sh + dd] for dd, wv in enumerate(wvs)]
        for dd in range(_DIM):
          plsc.store_scatter(o_v, [rowb, l9d[dd]], prods[dd])

  n_quads = (n_chunks + _NBUF - 1) // _NBUF

  for t in range(_NBUF):
    start_in(t, t)

  def main_body(k, carry):
    t0 = _NBUF * k
    for b in range(_NBUF):
      t = t0 + b

      @pl.when(t < n_chunks)
      def _():
        wait_in(b, t)

        @pl.when(t >= _NBUF)
        def _():
          wait_out(b, t - _NBUF)

        compute(b)
        start_out(b, t)

        @pl.when(t + _NBUF < n_chunks)
        def _():
          start_in(b, t + _NBUF)

    return carry

  lax.fori_loop(0, n_quads, main_body, 0)

  for t in range(n_chunks - _NBUF, n_chunks):
    wait_out(t % _NBUF, t)


@jax.jit
def _run(a2d, w2d):
  e_total = w2d.shape[0]
  n_chunks = e_total // (_NW * _CHUNK)
  mesh = plsc.VectorSubcoreMesh(core_axis_name="c", subcore_axis_name="s")
  body = functools.partial(_sc_body, n_chunks)
  sc_kernel = pl.kernel(
      body,
      out_type=jax.ShapeDtypeStruct((e_total, _OUTW), jnp.float32),
      mesh=mesh,
      compiler_params=pltpu.CompilerParams(
          needs_layout_passes=False, use_tc_tiling_on_sc=False),
      scratch_types=(
          [pltpu.VMEM((_GRP, _AROW), jnp.float32)] * _NBUF
          + [pltpu.VMEM((_CHUNK, _WW), jnp.float32)] * _NBUF
          + [pltpu.VMEM((_CHUNK, _OUTW), jnp.float32)] * _NBUF
          + [pltpu.SemaphoreType.DMA] * (2 * _NBUF)
      ),
  )
  return sc_kernel(a2d, w2d)


def kernel(edge_attr, weights):
  e = edge_attr.shape[0]
  assert e % (_NW * _CHUNK) == 0 and e // (_NW * _CHUNK) >= 2 * _NBUF, e
  out = _run(edge_attr.reshape(e // _LANES, _AROW), weights)
  return out.reshape(e, _MUL, _DIM)
